# Initial kernel scaffold; baseline (speedup 1.0000x reference)
import jax
import jax.numpy as jnp
from jax.experimental import pallas as pl


def kernel(history_timedeltas, history_weights, W_user, W_item, user_lamb, user_bias, item_bias, global_lamb, alpha, beta, gamma, cutoff, smooth, force, user_index, pred_item_indices, history_item_indices):
    raise NotImplementedError("write your pallas kernel here")


# speedup vs baseline: 1.1753x; 1.1753x over previous
import jax
import jax.numpy as jnp
from jax import lax
from jax.experimental import pallas as pl
from jax.experimental.pallas import tpu as pltpu
from jax.experimental.pallas import tpu_sc as plsc

B = 8
P = 1000
PP = 1024
H = 128
D = 64

NC = 2
NS = 16
NW = NC * NS

PRED_PER_W = (B * PP) // NW
PRED_CHUNKS = PRED_PER_W // 128
HIST_PER_W = (B * H) // NW


def _sc_gather_body(w_item, w_user, ib_tbl, ul_tbl, ub_tbl,
                    pred_idx, hist_idx, user_idx,
                    pred_out, hist_out, ib_out, user_out, ul_out, ub_out,
                    pidx_v, prow_v, pbias_v, hidx_v, hrow_v,
                    uidx_v, urow_v, ulv, ubv, sem, sem2):
    wid = lax.axis_index("s") * NC + lax.axis_index("c")

    pltpu.sync_copy(pred_idx.at[pl.ds(wid * PRED_CHUNKS, PRED_CHUNKS)], pidx_v)
    copies = []
    for j in range(PRED_CHUNKS):
        copies.append(pltpu.async_copy(
            w_item.at[pidx_v.at[j]], prow_v.at[pl.ds(j * 128, 128)], sem))
        copies.append(pltpu.async_copy(
            ib_tbl.at[pidx_v.at[j]], pbias_v.at[j], sem))

    pltpu.sync_copy(hist_idx.at[wid], hidx_v)
    copies.append(pltpu.async_copy(w_item.at[hidx_v], hrow_v, sem))
    for c in copies:
        c.wait()

    pltpu.sync_copy(prow_v, pred_out.at[pl.ds(wid * PRED_PER_W, PRED_PER_W)])
    pltpu.sync_copy(pbias_v, ib_out.at[pl.ds(wid * PRED_CHUNKS, PRED_CHUNKS)])
    pltpu.sync_copy(hrow_v, hist_out.at[pl.ds(wid * HIST_PER_W, HIST_PER_W)])

    @pl.when(wid == 0)
    def _():
        pltpu.sync_copy(user_idx, uidx_v)
        cu = pltpu.async_copy(w_user.at[uidx_v], urow_v, sem2)
        cl = pltpu.async_copy(ul_tbl.at[uidx_v], ulv, sem2)
        cb = pltpu.async_copy(ub_tbl.at[uidx_v], ubv, sem2)
        cu.wait()
        cl.wait()
        cb.wait()
        pltpu.sync_copy(urow_v, user_out)
        pltpu.sync_copy(ulv, ul_out)
        pltpu.sync_copy(ubv, ub_out)


def _make_sc_gather():
    return pl.kernel(
        _sc_gather_body,
        out_type=[
            jax.ShapeDtypeStruct((B * PP, D), jnp.float32),
            jax.ShapeDtypeStruct((B * H, D), jnp.float32),
            jax.ShapeDtypeStruct((NW * PRED_CHUNKS, 128), jnp.float32),
            jax.ShapeDtypeStruct((B, D), jnp.float32),
            jax.ShapeDtypeStruct((B,), jnp.float32),
            jax.ShapeDtypeStruct((B,), jnp.float32),
        ],
        mesh=plsc.VectorSubcoreMesh(core_axis_name="c", subcore_axis_name="s",
                                    num_cores=NC, num_subcores=NS),
        compiler_params=pltpu.CompilerParams(use_tc_tiling_on_sc=False),
        scratch_types=[
            pltpu.VMEM((PRED_CHUNKS, 128), jnp.int32),
            pltpu.VMEM((PRED_PER_W, D), jnp.float32),
            pltpu.VMEM((PRED_CHUNKS, 128), jnp.float32),
            pltpu.VMEM((HIST_PER_W,), jnp.int32),
            pltpu.VMEM((HIST_PER_W, D), jnp.float32),
            pltpu.VMEM((B,), jnp.int32),
            pltpu.VMEM((B, D), jnp.float32),
            pltpu.VMEM((B,), jnp.float32),
            pltpu.VMEM((B,), jnp.float32),
            pltpu.SemaphoreType.DMA,
            pltpu.SemaphoreType.DMA,
        ],
    )


def _tc_dense_body(pred_ref, hist_ref, user_ref, ib_ref, td_ref, wt_ref,
                   ul_ref, ub_ref, par_ref, out_ref):
    glamb = par_ref[0]
    alpha = par_ref[1]
    beta = par_ref[2]
    gamma = par_ref[3]
    cutoff = par_ref[4]
    smooth = par_ref[5]
    force = par_ref[6]
    inv_denom = 1.0 + jnp.exp(force * smooth - smooth)
    ones_row = jnp.ones((1, D), jnp.float32)
    dn_t = (((1,), (1,)), ((), ()))
    dn_s = (((1,), (0,)), ((), ()))
    for b in range(B):
        pred = pred_ref[b]
        hist = hist_ref[b]
        u = user_ref[b:b + 1, :]
        pn = lax.dot_general(ones_row, pred * pred, dn_t,
                             preferred_element_type=jnp.float32,
                             precision=lax.Precision.HIGHEST)
        ph = lax.dot_general(hist, pred, dn_t,
                             preferred_element_type=jnp.float32,
                             precision=lax.Precision.HIGHEST)
        up = lax.dot_general(u, pred, dn_t,
                             preferred_element_type=jnp.float32,
                             precision=lax.Precision.HIGHEST)
        hn = jnp.sum(hist * hist, axis=1, keepdims=True)
        un = jnp.sum(u * u, axis=1, keepdims=True)
        dist = jnp.sqrt(jnp.maximum(hn + pn - 2.0 * ph, 0.0))
        sig = inv_denom / (1.0 + jnp.exp(force * smooth - smooth / (1.0 + dist)))
        coeff = ((glamb + ul_ref[b])
                 * lax.rsqrt(td_ref[b:b + 1, :] + cutoff)
                 * wt_ref[b:b + 1, :])
        res = lax.dot_general(coeff, sig, dn_s,
                              preferred_element_type=jnp.float32,
                              precision=lax.Precision.HIGHEST)
        du = jnp.sqrt(jnp.maximum(un + pn - 2.0 * up, 0.0))
        outp = jnp.maximum(du - res, 0.0)
        out_ref[b:b + 1, :] = (alpha * outp + beta * outp * outp + gamma
                               + ub_ref[b] + ib_ref[b:b + 1, :])


def _tc_dense(pred3, hist3, user_rows, ib2, td, wt, ul_g, ub_g, params):
    return pl.pallas_call(
        _tc_dense_body,
        out_shape=jax.ShapeDtypeStruct((B, PP), jnp.float32),
        in_specs=[
            pl.BlockSpec(memory_space=pltpu.VMEM),
            pl.BlockSpec(memory_space=pltpu.VMEM),
            pl.BlockSpec(memory_space=pltpu.VMEM),
            pl.BlockSpec(memory_space=pltpu.VMEM),
            pl.BlockSpec(memory_space=pltpu.VMEM),
            pl.BlockSpec(memory_space=pltpu.VMEM),
            pl.BlockSpec(memory_space=pltpu.SMEM),
            pl.BlockSpec(memory_space=pltpu.SMEM),
            pl.BlockSpec(memory_space=pltpu.SMEM),
        ],
        out_specs=pl.BlockSpec(memory_space=pltpu.VMEM),
    )(pred3, hist3, user_rows, ib2, td, wt, ul_g, ub_g, params)


def kernel(history_timedeltas, history_weights, W_user, W_item, user_lamb,
           user_bias, item_bias, global_lamb, alpha, beta, gamma, cutoff,
           smooth, force, user_index, pred_item_indices,
           history_item_indices):
    pidx = jnp.pad(pred_item_indices.astype(jnp.int32),
                   ((0, 0), (0, PP - P))).reshape(NW * PRED_CHUNKS, 128)
    hidx = history_item_indices.astype(jnp.int32).reshape(NW, HIST_PER_W)
    uidx = user_index.astype(jnp.int32)
    pred_rows, hist_rows, ib_g, user_rows, ul_g, ub_g = _make_sc_gather()(
        W_item, W_user, item_bias.reshape(-1), user_lamb.reshape(-1),
        user_bias.reshape(-1), pidx, hidx, uidx)
    params = jnp.stack([global_lamb, alpha, beta, gamma, cutoff, smooth,
                        force]).astype(jnp.float32)
    I_full = _tc_dense(pred_rows.reshape(B, PP, D), hist_rows.reshape(B, H, D),
                       user_rows, ib_g.reshape(B, PP), history_timedeltas,
                       history_weights, ul_g, ub_g, params)
    return I_full[:, :P]
